# baseline (speedup 1.0000x reference)
import jax
import jax.numpy as jnp
from jax import lax
from jax.experimental import pallas as pl

P = 8
PP = P * P
H = 16
B_BLK = 4
C_BLK = 16


def _halo_kernel(tw_ref, bw_ref, lw_ref, rw_ref, tlw_ref, trw_ref,
                 blw_ref, brw_ref, x_ref, out_ref):
    x = x_ref[...]
    b0 = pl.program_id(0) * B_BLK
    i = b0 + lax.broadcasted_iota(jnp.int32, (B_BLK, 1, 1, 1), 0)
    im = i % PP
    r = im // P
    c = im % P
    mT = r > 0
    mB = r < P - 1
    mL = c > 0
    mR = c < P - 1

    tW = tw_ref[...]
    bW = bw_ref[...]
    lW = lw_ref[...]
    rW = rw_ref[...]

    zero = jnp.zeros((), jnp.float32)
    top = jnp.where(mT, tW * x[:, :, 0:1, :] + (1.0 - tW) * x[:, :, 1:2, :], zero)
    bot = jnp.where(mB, bW * x[:, :, H-1:H, :] + (1.0 - bW) * x[:, :, H-2:H-1, :], zero)
    left = jnp.where(mL, lW * x[:, :, :, 0:1] + (1.0 - lW) * x[:, :, :, 1:2], zero)
    right = jnp.where(mR, rW * x[:, :, :, H-1:H] + (1.0 - rW) * x[:, :, :, H-2:H-1], zero)
    tl = jnp.where(mT & mL, tlw_ref[...] * x[:, :, 0:1, 0:1], zero)
    tr = jnp.where(mT & mR, trw_ref[...] * x[:, :, 0:1, H-1:H], zero)
    bl = jnp.where(mB & mL, blw_ref[...] * x[:, :, H-1:H, 0:1], zero)
    br = jnp.where(mB & mR, brw_ref[...] * x[:, :, H-1:H, H-1:H], zero)

    out_ref[:, :, 1:H+1, 1:H+1] = x
    out_ref[:, :, 0:1, 1:H+1] = top
    out_ref[:, :, H+1:H+2, 1:H+1] = bot
    out_ref[:, :, 1:H+1, 0:1] = left
    out_ref[:, :, 1:H+1, H+1:H+2] = right
    out_ref[:, :, 0:1, 0:1] = tl
    out_ref[:, :, 0:1, H+1:H+2] = tr
    out_ref[:, :, H+1:H+2, 0:1] = bl
    out_ref[:, :, H+1:H+2, H+1:H+2] = br


def kernel(x, topW, botW, leftW, rightW, topleftW, toprightW, botleftW,
           botrightW, padding, num_patches, scaling_factor):
    b, C, ph, pw = x.shape
    ws = [(2.0 * jnp.tanh(w / 2.0)).reshape(1, C, 1, 1)
          for w in (topW, botW, leftW, rightW,
                    topleftW, toprightW, botleftW, botrightW)]

    w_spec = pl.BlockSpec((1, C_BLK, 1, 1), lambda i, j: (0, j, 0, 0))
    out = pl.pallas_call(
        _halo_kernel,
        grid=(b // B_BLK, C // C_BLK),
        in_specs=[w_spec] * 8 + [
            pl.BlockSpec((B_BLK, C_BLK, ph, pw), lambda i, j: (i, j, 0, 0)),
        ],
        out_specs=pl.BlockSpec((B_BLK, C_BLK, ph + 2, pw + 2),
                               lambda i, j: (i, j, 0, 0)),
        out_shape=jax.ShapeDtypeStruct((b, C, ph + 2, pw + 2), x.dtype),
    )(*ws, x)
    return out


# speedup vs baseline: 18.7542x; 18.7542x over previous
import jax
import jax.numpy as jnp
from jax import lax
from jax.experimental import pallas as pl

P = 8
PP = P * P
H = 16
C_BLK = 8
R_OUT = 9
R_IN = 8


def _halo_kernel(tw_ref, bw_ref, lw_ref, rw_ref, tlw_ref, trw_ref,
                 blw_ref, brw_ref, x_ref, out_ref):
    k = pl.program_id(0)

    bm = lax.broadcasted_iota(jnp.int32, (1, 1, C_BLK, 1024), 3)
    im = bm % PP
    r = im // P
    c = im % P
    mT = r > 0
    mB = r < P - 1
    mL = c > 0
    mR = c < P - 1

    tW = tw_ref[...]
    bW = bw_ref[...]
    lW = lw_ref[...]
    rW = rw_ref[...]
    zero = jnp.zeros((), jnp.float32)

    def interior(j0, rr):
        xr = x_ref[pl.ds(rr, 1)]
        c0 = jnp.where(mL, lW * xr[:, 0:1] + (1.0 - lW) * xr[:, 1:2], zero)
        c17 = jnp.where(mR, rW * xr[:, H-1:H] + (1.0 - rW) * xr[:, H-2:H-1],
                        zero)
        out_ref[j0:j0+1] = jnp.concatenate([c0, xr, c17], axis=1)

    @pl.when(k == 0)
    def _():
        x0 = x_ref[0:1]
        x1 = x_ref[1:2]
        mid = jnp.where(mT, tW * x0 + (1.0 - tW) * x1, zero)
        c0 = jnp.where(mT & mL, tlw_ref[...] * x0[:, 0:1], zero)
        c17 = jnp.where(mT & mR, trw_ref[...] * x0[:, H-1:H], zero)
        out_ref[0:1] = jnp.concatenate([c0, mid, c17], axis=1)

    @pl.when(k > 0)
    def _():
        interior(0, k - 1)

    for j0 in range(1, R_OUT - 1):
        interior(j0, k + j0 - 1)

    @pl.when(k == 0)
    def _():
        interior(R_OUT - 1, R_OUT - 2)

    @pl.when(k > 0)
    def _():
        x15 = x_ref[R_IN-1:R_IN]
        x14 = x_ref[R_IN-2:R_IN-1]
        mid = jnp.where(mB, bW * x15 + (1.0 - bW) * x14, zero)
        c0 = jnp.where(mB & mL, blw_ref[...] * x15[:, 0:1], zero)
        c17 = jnp.where(mB & mR, brw_ref[...] * x15[:, H-1:H], zero)
        out_ref[R_OUT-1:R_OUT] = jnp.concatenate([c0, mid, c17], axis=1)


def kernel(x, topW, botW, leftW, rightW, topleftW, toprightW, botleftW,
           botrightW, padding, num_patches, scaling_factor):
    b, C, ph, pw = x.shape
    xt = jnp.transpose(x, (2, 3, 1, 0))
    ws = [jnp.broadcast_to((2.0 * jnp.tanh(w / 2.0)).reshape(1, 1, C, 1),
                           (1, 1, C, b))
          for w in (topW, botW, leftW, rightW,
                    topleftW, toprightW, botleftW, botrightW)]

    w_spec = pl.BlockSpec((1, 1, C_BLK, b), lambda k, j: (0, 0, j, 0))
    out_t = pl.pallas_call(
        _halo_kernel,
        grid=(2, C // C_BLK),
        in_specs=[w_spec] * 8 + [
            pl.BlockSpec((R_IN, pw, C_BLK, b), lambda k, j: (k, 0, j, 0)),
        ],
        out_specs=pl.BlockSpec((R_OUT, pw + 2, C_BLK, b),
                               lambda k, j: (k, 0, j, 0)),
        out_shape=jax.ShapeDtypeStruct((ph + 2, pw + 2, C, b), x.dtype),
    )(*ws, xt)
    return jnp.transpose(out_t, (3, 2, 0, 1))
